# baseline (speedup 1.0000x reference)
import jax
import jax.numpy as jnp
from jax import lax
from jax.experimental import pallas as pl
from jax.experimental.pallas import tpu as pltpu
from jax.experimental.pallas import tpu_sc as plsc

N_ENT = 1000000
N_REL = 1000
DIM = 64
B = 16384

NC = 2
NS = 16
L = 16
NW = NC * NS
BPW = B // NW
CH = 128
NCH = BPW // CH
K = DIM // L

NG = (N_ENT + 127) // 128
GPW = NG // NW
GREM = NG - GPW * NW
NBK = 16
MCAP = B + L
SROWS = 32
FLUSH_AT = SROWS - L


def _bc(x):
    return lax.broadcast(x, (L,))


def _hsum(v):
    return _bc(jnp.sum(v))


def _rsqrt_vec(x):
    xi = lax.bitcast_convert_type(x, jnp.int32)
    yi = jnp.int32(0x5F3759DF) - lax.shift_right_arithmetic(xi, jnp.int32(1))
    y = lax.bitcast_convert_type(yi, jnp.float32)
    xh = x * 0.5
    for _ in range(3):
        y = y * (1.5 - xh * y * y)
    return y


def _pop(m):
    return plsc.all_reduce_population_count(m)[0]


def _gather_body(emb_t, tr_t, emb_tl, tr_tl, ph, pt, ghe, ght, gte, gtt,
                 idxv, mb_h, mb_t, w_bufs, stg_h, stg_t,
                 sidx_h, sidx_t, sems, ssem):
    c = lax.axis_index("c")
    s = lax.axis_index("s")
    w = s * NC + c
    lo = w * GPW + jnp.minimum(w, GREM)
    ngr = GPW + (w < GREM).astype(jnp.int32)

    lane = lax.iota(jnp.int32, L)

    def scan_list(src_hbm, mbuf):
        pltpu.sync_copy(src_hbm, idxv.at[pl.ds(0, B)])

        def body(v, cnt):
            e = idxv[pl.ds(v * L, L)]
            eg = lax.shift_right_logical(e, 7)
            egl = eg - lo
            m = (egl >= 0) & (egl < ngr)
            packed = ((egl << 21) | ((e & 127) << 14) | (v * L + lane))
            plsc.store_compressed(mbuf.at[pl.ds(cnt, L)], packed, mask=m)
            return cnt + _pop(m)

        return lax.fori_loop(0, B // L, body, jnp.int32(0))

    cnt_h = scan_list(ph, mb_h)
    cnt_t = scan_list(pt, mb_t)

    def bucketize(mbuf, cnt, dst):
        nv = (cnt + L - 1) >> 4

        def count_body(u, counts):
            pk = mbuf[pl.ds(u * L, L)]
            valid = (u * L + lane) < cnt
            bk = lax.shift_right_logical(pk, 25)
            new = []
            for b in range(NBK):
                new.append(counts[b] + _pop((bk == b) & valid))
            return tuple(new)

        counts = lax.fori_loop(0, nv, count_body,
                               tuple(jnp.int32(0) for _ in range(NBK)))
        starts = []
        acc = jnp.int32(0)
        for b in range(NBK):
            starts.append(acc)
            acc = acc + counts[b]

        def place_body(u, offs):
            pk = mbuf[pl.ds(u * L, L)]
            valid = (u * L + lane) < cnt
            bk = lax.shift_right_logical(pk, 25)
            new = []
            for b in range(NBK):
                m = (bk == b) & valid
                plsc.store_compressed(dst.at[pl.ds(offs[b], L)], pk, mask=m)
                new.append(offs[b] + _pop(m))
            return tuple(new)

        lax.fori_loop(0, nv, place_body, tuple(starts))
        return starts, counts

    st_h, ct_h = bucketize(mb_h, cnt_h, idxv)
    bk_h = idxv
    st_t, ct_t = bucketize(mb_t, cnt_t, mb_h)
    bk_t = mb_h

    def issue_window(gr, par):
        embw, trw = w_bufs[par]
        sem = sems[par]
        g = lo + gr
        col = pl.multiple_of(g * 128, 128)

        @pl.when(g < NG - 1)
        def _():
            pltpu.async_copy(emb_t.at[:, pl.ds(col, 128)], embw, sem)
            pltpu.async_copy(tr_t.at[:, pl.ds(col, 128)], trw, sem)

        @pl.when(g == NG - 1)
        def _():
            pltpu.async_copy(emb_tl, embw, sem)
            pltpu.async_copy(tr_tl, trw, sem)

    def wait_window(gr, par):
        embw, trw = w_bufs[par]
        sem = sems[par]
        pltpu.make_async_copy(emb_t.at[:, pl.ds(0, 128)], embw, sem).wait()
        pltpu.make_async_copy(emb_t.at[:, pl.ds(0, 128)], trw, sem).wait()

    def flush(stg, sidx, out_a, out_b, fill):
        for v in range(SROWS // L):
            cur = sidx[pl.ds(v * L, L)]
            pad = (v * L + lane) >= fill
            sidx[pl.ds(v * L, L)] = jnp.where(pad, B + lane, cur)
        sa, sb = stg
        ca = pltpu.async_copy(sa, out_a.at[sidx], ssem)
        cb = pltpu.async_copy(sb, out_b.at[sidx], ssem)
        ca.wait()
        cb.wait()

    def select16(vals, b):
        r = vals[0]
        for i in range(1, NBK):
            r = jnp.where(b == i, vals[i], r)
        return r

    def process_window(gr, par, bk, starts, counts, stg, sidx, out_a, out_b,
                       fill):
        embw, trw = w_bufs[par]
        bkt = lax.shift_right_logical(gr, 4)
        start = select16(starts, bkt)
        cntb = select16(counts, bkt)
        nv = (cntb + L - 1) >> 4

        def body(u, fill):
            off = start + u * L
            pk = bk[pl.ds(off, L)]
            valid = (off + lane) < (start + cntb)
            mm = (lax.shift_right_logical(pk, 21) == gr) & valid
            nm = _pop(mm)

            def matched(fill):
                ein = lax.shift_right_logical(pk, 14) & 127
                slot = pk & 16383
                mmi = jnp.where(mm, 1, 0)
                ranks = plsc.cumsum(mmi) - 1
                plsc.store_compressed(sidx.at[pl.ds(fill, L)], slot, mask=mm)
                sa, sb = stg
                for j in range(L):
                    @pl.when(mmi[j] > 0)
                    def _():
                        ej = _bc(ein[j])
                        row = fill + ranks[j]
                        for k in range(K):
                            fidx = lane + (k * L)
                            sa[row, pl.ds(k * L, L)] = plsc.load_gather(
                                embw, [fidx, ej])
                            sb[row, pl.ds(k * L, L)] = plsc.load_gather(
                                trw, [fidx, ej])
                return fill + nm

            fill = lax.cond(nm > 0, matched, lambda f: f, fill)

            def do_flush(f):
                flush(stg, sidx, out_a, out_b, f)
                return jnp.int32(0)

            fill = lax.cond(fill >= FLUSH_AT, do_flush, lambda f: f, fill)
            return fill

        return lax.fori_loop(0, nv, body, fill)

    issue_window(0, 0)

    @pl.when(ngr > 1)
    def _():
        issue_window(1, 1)

    ntrip = (ngr + 2) // 3

    def trip_body(p, carry):
        fh, ft = carry
        for j in range(3):
            g = 3 * p + j

            @pl.when(g + 2 < ngr)
            def _(g=g, j=j):
                issue_window(g + 2, (j + 2) % 3)

            def do(carry, g=g, j=j):
                fh, ft = carry
                wait_window(g, j)
                fh = process_window(g, j, bk_h, st_h, ct_h, stg_h, sidx_h,
                                    ghe, ght, fh)
                ft = process_window(g, j, bk_t, st_t, ct_t, stg_t, sidx_t,
                                    gte, gtt, ft)
                return fh, ft

            fh, ft = lax.cond(g < ngr, do, lambda cc: cc, (fh, ft))
        return fh, ft

    fh, ft = lax.fori_loop(0, ntrip, trip_body, (jnp.int32(0), jnp.int32(0)))
    flush(stg_h, sidx_h, ghe, ght, fh)
    flush(stg_t, sidx_t, gte, gtt, ft)


def _score_body(ghe, ght, gte, gtt, rel_emb, rel_tr, pr, out,
                idx_r, hb, htb, tb, ttb, rb, rtb, outv, sem):
    c = lax.axis_index("c")
    s = lax.axis_index("s")
    wid = s * NC + c
    base = wid * BPW

    pltpu.sync_copy(pr.at[wid], idx_r)

    lane = lax.iota(jnp.int32, L)
    mask0 = lane == 0

    for ci in range(NCH):
        row0 = base + ci * CH
        cps = [
            pltpu.async_copy(ghe.at[pl.ds(row0, CH), pl.ds(0, DIM)], hb, sem),
            pltpu.async_copy(ght.at[pl.ds(row0, CH), pl.ds(0, DIM)], htb, sem),
            pltpu.async_copy(gte.at[pl.ds(row0, CH), pl.ds(0, DIM)], tb, sem),
            pltpu.async_copy(gtt.at[pl.ds(row0, CH), pl.ds(0, DIM)], ttb, sem),
            pltpu.async_copy(rel_emb.at[idx_r.at[ci]], rb, sem),
            pltpu.async_copy(rel_tr.at[idx_r.at[ci]], rtb, sem),
        ]
        for cp in cps:
            cp.wait()

        @plsc.parallel_loop(0, CH, 1, unroll=2)
        def row_body(i):
            h = [hb[i, pl.ds(L * k, L)] for k in range(K)]
            ht = [htb[i, pl.ds(L * k, L)] for k in range(K)]
            t = [tb[i, pl.ds(L * k, L)] for k in range(K)]
            tt = [ttb[i, pl.ds(L * k, L)] for k in range(K)]
            r = [rb[i, pl.ds(L * k, L)] for k in range(K)]
            rt = [rtb[i, pl.ds(L * k, L)] for k in range(K)]

            def dot(a, b):
                p = a[0] * b[0]
                for k in range(1, K):
                    p = p + a[k] * b[k]
                return _hsum(p)

            d_hht = dot(h, ht)
            d_hrt = dot(h, rt)
            d_rtrt = dot(rt, rt)
            d_hh = dot(h, h)
            d_ttt = dot(t, tt)
            d_trt = dot(t, rt)
            d_tt = dot(t, t)

            n_h = d_hh + d_hht * (2.0 * d_hrt + d_hht * d_rtrt)
            n_t = d_tt + d_ttt * (2.0 * d_trt + d_ttt * d_rtrt)
            rih = _rsqrt_vec(jnp.maximum(n_h, 1e-12))
            rit = _rsqrt_vec(jnp.maximum(n_t, 1e-12))

            bmix = d_hht * rih - d_ttt * rit
            q = jnp.abs(h[0] * rih + rt[0] * bmix + r[0] - t[0] * rit)
            for k in range(1, K):
                q = q + jnp.abs(h[k] * rih + rt[k] * bmix + r[k] - t[k] * rit)
            plsc.store_scatter(
                outv, [_bc(jnp.int32(ci * CH) + i)], _hsum(q), mask=mask0)

    pltpu.sync_copy(outv, out.at[wid])


@jax.jit
def _run(ph, pt, pr, ent_emb, rel_emb, ent_tr, rel_tr):
    mesh = plsc.VectorSubcoreMesh(core_axis_name="c", subcore_axis_name="s")
    gshape = jax.ShapeDtypeStruct((B + L, 128), jnp.float32)
    gather = pl.kernel(
        _gather_body,
        out_type=(gshape, gshape, gshape, gshape),
        mesh=mesh,
        scratch_types=[
            pltpu.VMEM((MCAP,), jnp.int32),
            pltpu.VMEM((MCAP,), jnp.int32),
            pltpu.VMEM((MCAP,), jnp.int32),
            [[pltpu.VMEM((DIM, 128), jnp.float32) for _ in range(2)]
             for _ in range(3)],
            [pltpu.VMEM((SROWS, 128), jnp.float32) for _ in range(2)],
            [pltpu.VMEM((SROWS, 128), jnp.float32) for _ in range(2)],
            pltpu.VMEM((SROWS,), jnp.int32),
            pltpu.VMEM((SROWS,), jnp.int32),
            [pltpu.SemaphoreType.DMA, pltpu.SemaphoreType.DMA,
             pltpu.SemaphoreType.DMA],
            pltpu.SemaphoreType.DMA,
        ],
        compiler_params=pltpu.CompilerParams(
            needs_layout_passes=False, use_tc_tiling_on_sc=True),
    )
    emb_view = ent_emb.T
    tr_view = ent_tr.T
    ghe, ght, gte, gtt = gather(
        emb_view, tr_view,
        lax.pad(lax.slice(emb_view, (0, N_ENT - 64), (DIM, N_ENT)),
                0.0, ((0, 0, 0), (0, 64, 0))),
        lax.pad(lax.slice(tr_view, (0, N_ENT - 64), (DIM, N_ENT)),
                0.0, ((0, 0, 0), (0, 64, 0))),
        ph, pt)

    score = pl.kernel(
        _score_body,
        out_type=jax.ShapeDtypeStruct((NW, BPW), jnp.float32),
        mesh=mesh,
        scratch_types=[
            pltpu.VMEM((NCH, CH), jnp.int32),
            pltpu.VMEM((CH, DIM), jnp.float32),
            pltpu.VMEM((CH, DIM), jnp.float32),
            pltpu.VMEM((CH, DIM), jnp.float32),
            pltpu.VMEM((CH, DIM), jnp.float32),
            pltpu.VMEM((CH, DIM), jnp.float32),
            pltpu.VMEM((CH, DIM), jnp.float32),
            pltpu.VMEM((BPW,), jnp.float32),
            pltpu.SemaphoreType.DMA,
        ],
        compiler_params=pltpu.CompilerParams(
            needs_layout_passes=False, use_tc_tiling_on_sc=False),
    )
    out = score(ghe, ght, gte, gtt, rel_emb, rel_tr,
                pr.reshape(NW, NCH, CH))
    return out.reshape(B, 1)


def kernel(predict_h, predict_t, predict_r, ent_embeddings, rel_embeddings,
           ent_transfer, rel_transfer):
    return _run(predict_h, predict_t, predict_r, ent_embeddings,
                rel_embeddings, ent_transfer, rel_transfer)


# speedup vs baseline: 1.0047x; 1.0047x over previous
import jax
import jax.numpy as jnp
from jax import lax
from jax.experimental import pallas as pl
from jax.experimental.pallas import tpu as pltpu
from jax.experimental.pallas import tpu_sc as plsc

N_ENT = 1000000
N_REL = 1000
DIM = 64
B = 16384

NC = 2
NS = 16
L = 16
NW = NC * NS
BPW = B // NW
CH = 128
NCH = BPW // CH
K = DIM // L

NG = (N_ENT + 127) // 128
GPW = NG // NW
GREM = NG - GPW * NW
NBK = 16
MCAP = B + L
SROWS = 48
FLUSH_AT = SROWS - L


def _bc(x):
    return lax.broadcast(x, (L,))


def _hsum(v):
    return _bc(jnp.sum(v))


def _rsqrt_vec(x):
    xi = lax.bitcast_convert_type(x, jnp.int32)
    yi = jnp.int32(0x5F3759DF) - lax.shift_right_arithmetic(xi, jnp.int32(1))
    y = lax.bitcast_convert_type(yi, jnp.float32)
    xh = x * 0.5
    for _ in range(3):
        y = y * (1.5 - xh * y * y)
    return y


def _pop(m):
    return plsc.all_reduce_population_count(m)[0]


def _gather_body(emb_t, tr_t, emb_tl, tr_tl, ph, pt, ghe, ght, gte, gtt,
                 idxv, mb_h, mb_t, w_bufs, stg_h, stg_t,
                 sidx_h, sidx_t, sems, ssem):
    c = lax.axis_index("c")
    s = lax.axis_index("s")
    w = s * NC + c
    lo = w * GPW + jnp.minimum(w, GREM)
    ngr = GPW + (w < GREM).astype(jnp.int32)

    lane = lax.iota(jnp.int32, L)

    def scan_list(src_hbm, mbuf):
        pltpu.sync_copy(src_hbm, idxv.at[pl.ds(0, B)])

        def body(v, cnt):
            e = idxv[pl.ds(v * L, L)]
            eg = lax.shift_right_logical(e, 7)
            egl = eg - lo
            m = (egl >= 0) & (egl < ngr)
            packed = ((egl << 21) | ((e & 127) << 14) | (v * L + lane))
            plsc.store_compressed(mbuf.at[pl.ds(cnt, L)], packed, mask=m)
            return cnt + _pop(m)

        return lax.fori_loop(0, B // L, body, jnp.int32(0))

    cnt_h = scan_list(ph, mb_h)
    cnt_t = scan_list(pt, mb_t)

    def bucketize(mbuf, cnt, dst):
        nv = (cnt + L - 1) >> 4

        def count_body(u, counts):
            pk = mbuf[pl.ds(u * L, L)]
            valid = (u * L + lane) < cnt
            bk = lax.shift_right_logical(pk, 25)
            new = []
            for b in range(NBK):
                new.append(counts[b] + _pop((bk == b) & valid))
            return tuple(new)

        counts = lax.fori_loop(0, nv, count_body,
                               tuple(jnp.int32(0) for _ in range(NBK)))
        starts = []
        acc = jnp.int32(0)
        for b in range(NBK):
            starts.append(acc)
            acc = acc + counts[b]

        def place_body(u, offs):
            pk = mbuf[pl.ds(u * L, L)]
            valid = (u * L + lane) < cnt
            bk = lax.shift_right_logical(pk, 25)
            new = []
            for b in range(NBK):
                m = (bk == b) & valid
                plsc.store_compressed(dst.at[pl.ds(offs[b], L)], pk, mask=m)
                new.append(offs[b] + _pop(m))
            return tuple(new)

        lax.fori_loop(0, nv, place_body, tuple(starts))
        return starts, counts

    st_h, ct_h = bucketize(mb_h, cnt_h, idxv)
    bk_h = idxv
    st_t, ct_t = bucketize(mb_t, cnt_t, mb_h)
    bk_t = mb_h

    def issue_window(gr, par):
        embw, trw = w_bufs[par]
        sem = sems[par]
        g = lo + gr
        col = pl.multiple_of(g * 128, 128)

        @pl.when(g < NG - 1)
        def _():
            pltpu.async_copy(emb_t.at[:, pl.ds(col, 128)], embw, sem)
            pltpu.async_copy(tr_t.at[:, pl.ds(col, 128)], trw, sem)

        @pl.when(g == NG - 1)
        def _():
            pltpu.async_copy(emb_tl, embw, sem)
            pltpu.async_copy(tr_tl, trw, sem)

    def wait_window(gr, par):
        embw, trw = w_bufs[par]
        sem = sems[par]
        pltpu.make_async_copy(emb_t.at[:, pl.ds(0, 128)], embw, sem).wait()
        pltpu.make_async_copy(emb_t.at[:, pl.ds(0, 128)], trw, sem).wait()

    def flush(stg, sidx, out_a, out_b, fill):
        for v in range(SROWS // L):
            cur = sidx[pl.ds(v * L, L)]
            pad = (v * L + lane) >= fill
            sidx[pl.ds(v * L, L)] = jnp.where(pad, B + lane, cur)
        sa, sb = stg
        ca = pltpu.async_copy(sa, out_a.at[sidx], ssem)
        cb = pltpu.async_copy(sb, out_b.at[sidx], ssem)
        ca.wait()
        cb.wait()

    def select16(vals, b):
        r = vals[0]
        for i in range(1, NBK):
            r = jnp.where(b == i, vals[i], r)
        return r

    def process_window(gr, par, bk, starts, counts, stg, sidx, out_a, out_b,
                       fill):
        embw, trw = w_bufs[par]
        bkt = lax.shift_right_logical(gr, 4)
        start = select16(starts, bkt)
        cntb = select16(counts, bkt)
        nv = (cntb + L - 1) >> 4

        def body(u, fill):
            off = start + u * L
            pk = bk[pl.ds(off, L)]
            valid = (off + lane) < (start + cntb)
            mm = (lax.shift_right_logical(pk, 21) == gr) & valid
            nm = _pop(mm)

            def matched(fill):
                ein = lax.shift_right_logical(pk, 14) & 127
                slot = pk & 16383
                mmi = jnp.where(mm, 1, 0)
                ranks = plsc.cumsum(mmi) - 1
                plsc.store_compressed(sidx.at[pl.ds(fill, L)], slot, mask=mm)
                sa, sb = stg
                for j in range(L):
                    @pl.when(mmi[j] > 0)
                    def _():
                        ej = _bc(ein[j])
                        row = fill + ranks[j]
                        for k in range(K):
                            fidx = lane + (k * L)
                            sa[row, pl.ds(k * L, L)] = plsc.load_gather(
                                embw, [fidx, ej])
                            sb[row, pl.ds(k * L, L)] = plsc.load_gather(
                                trw, [fidx, ej])
                return fill + nm

            fill = lax.cond(nm > 0, matched, lambda f: f, fill)

            def do_flush(f):
                flush(stg, sidx, out_a, out_b, f)
                return jnp.int32(0)

            fill = lax.cond(fill >= FLUSH_AT, do_flush, lambda f: f, fill)
            return fill

        return lax.fori_loop(0, nv, body, fill)

    issue_window(0, 0)

    @pl.when(ngr > 1)
    def _():
        issue_window(1, 1)

    ntrip = (ngr + 2) // 3

    def trip_body(p, carry):
        fh, ft = carry
        for j in range(3):
            g = 3 * p + j

            @pl.when(g + 2 < ngr)
            def _(g=g, j=j):
                issue_window(g + 2, (j + 2) % 3)

            def do(carry, g=g, j=j):
                fh, ft = carry
                wait_window(g, j)
                fh = process_window(g, j, bk_h, st_h, ct_h, stg_h, sidx_h,
                                    ghe, ght, fh)
                ft = process_window(g, j, bk_t, st_t, ct_t, stg_t, sidx_t,
                                    gte, gtt, ft)
                return fh, ft

            fh, ft = lax.cond(g < ngr, do, lambda cc: cc, (fh, ft))
        return fh, ft

    fh, ft = lax.fori_loop(0, ntrip, trip_body, (jnp.int32(0), jnp.int32(0)))
    flush(stg_h, sidx_h, ghe, ght, fh)
    flush(stg_t, sidx_t, gte, gtt, ft)


def _score_body(ghe, ght, gte, gtt, rel_emb, rel_tr, pr, out,
                idx_r, hb, htb, tb, ttb, rb, rtb, outv, sem):
    c = lax.axis_index("c")
    s = lax.axis_index("s")
    wid = s * NC + c
    base = wid * BPW

    pltpu.sync_copy(pr.at[wid], idx_r)

    lane = lax.iota(jnp.int32, L)
    mask0 = lane == 0

    for ci in range(NCH):
        row0 = base + ci * CH
        cps = [
            pltpu.async_copy(ghe.at[pl.ds(row0, CH), pl.ds(0, DIM)], hb, sem),
            pltpu.async_copy(ght.at[pl.ds(row0, CH), pl.ds(0, DIM)], htb, sem),
            pltpu.async_copy(gte.at[pl.ds(row0, CH), pl.ds(0, DIM)], tb, sem),
            pltpu.async_copy(gtt.at[pl.ds(row0, CH), pl.ds(0, DIM)], ttb, sem),
            pltpu.async_copy(rel_emb.at[idx_r.at[ci]], rb, sem),
            pltpu.async_copy(rel_tr.at[idx_r.at[ci]], rtb, sem),
        ]
        for cp in cps:
            cp.wait()

        @plsc.parallel_loop(0, CH, 1, unroll=2)
        def row_body(i):
            h = [hb[i, pl.ds(L * k, L)] for k in range(K)]
            ht = [htb[i, pl.ds(L * k, L)] for k in range(K)]
            t = [tb[i, pl.ds(L * k, L)] for k in range(K)]
            tt = [ttb[i, pl.ds(L * k, L)] for k in range(K)]
            r = [rb[i, pl.ds(L * k, L)] for k in range(K)]
            rt = [rtb[i, pl.ds(L * k, L)] for k in range(K)]

            def dot(a, b):
                p = a[0] * b[0]
                for k in range(1, K):
                    p = p + a[k] * b[k]
                return _hsum(p)

            d_hht = dot(h, ht)
            d_hrt = dot(h, rt)
            d_rtrt = dot(rt, rt)
            d_hh = dot(h, h)
            d_ttt = dot(t, tt)
            d_trt = dot(t, rt)
            d_tt = dot(t, t)

            n_h = d_hh + d_hht * (2.0 * d_hrt + d_hht * d_rtrt)
            n_t = d_tt + d_ttt * (2.0 * d_trt + d_ttt * d_rtrt)
            rih = _rsqrt_vec(jnp.maximum(n_h, 1e-12))
            rit = _rsqrt_vec(jnp.maximum(n_t, 1e-12))

            bmix = d_hht * rih - d_ttt * rit
            q = jnp.abs(h[0] * rih + rt[0] * bmix + r[0] - t[0] * rit)
            for k in range(1, K):
                q = q + jnp.abs(h[k] * rih + rt[k] * bmix + r[k] - t[k] * rit)
            plsc.store_scatter(
                outv, [_bc(jnp.int32(ci * CH) + i)], _hsum(q), mask=mask0)

    pltpu.sync_copy(outv, out.at[wid])


@jax.jit
def _run(ph, pt, pr, ent_emb, rel_emb, ent_tr, rel_tr):
    mesh = plsc.VectorSubcoreMesh(core_axis_name="c", subcore_axis_name="s")
    gshape = jax.ShapeDtypeStruct((B + L, 128), jnp.float32)
    gather = pl.kernel(
        _gather_body,
        out_type=(gshape, gshape, gshape, gshape),
        mesh=mesh,
        scratch_types=[
            pltpu.VMEM((MCAP,), jnp.int32),
            pltpu.VMEM((MCAP,), jnp.int32),
            pltpu.VMEM((MCAP,), jnp.int32),
            [[pltpu.VMEM((DIM, 128), jnp.float32) for _ in range(2)]
             for _ in range(3)],
            [pltpu.VMEM((SROWS, 128), jnp.float32) for _ in range(2)],
            [pltpu.VMEM((SROWS, 128), jnp.float32) for _ in range(2)],
            pltpu.VMEM((SROWS,), jnp.int32),
            pltpu.VMEM((SROWS,), jnp.int32),
            [pltpu.SemaphoreType.DMA, pltpu.SemaphoreType.DMA,
             pltpu.SemaphoreType.DMA],
            pltpu.SemaphoreType.DMA,
        ],
        compiler_params=pltpu.CompilerParams(
            needs_layout_passes=False, use_tc_tiling_on_sc=True),
    )
    emb_view = ent_emb.T
    tr_view = ent_tr.T
    ghe, ght, gte, gtt = gather(
        emb_view, tr_view,
        lax.pad(lax.slice(emb_view, (0, N_ENT - 64), (DIM, N_ENT)),
                0.0, ((0, 0, 0), (0, 64, 0))),
        lax.pad(lax.slice(tr_view, (0, N_ENT - 64), (DIM, N_ENT)),
                0.0, ((0, 0, 0), (0, 64, 0))),
        ph, pt)

    score = pl.kernel(
        _score_body,
        out_type=jax.ShapeDtypeStruct((NW, BPW), jnp.float32),
        mesh=mesh,
        scratch_types=[
            pltpu.VMEM((NCH, CH), jnp.int32),
            pltpu.VMEM((CH, DIM), jnp.float32),
            pltpu.VMEM((CH, DIM), jnp.float32),
            pltpu.VMEM((CH, DIM), jnp.float32),
            pltpu.VMEM((CH, DIM), jnp.float32),
            pltpu.VMEM((CH, DIM), jnp.float32),
            pltpu.VMEM((CH, DIM), jnp.float32),
            pltpu.VMEM((BPW,), jnp.float32),
            pltpu.SemaphoreType.DMA,
        ],
        compiler_params=pltpu.CompilerParams(
            needs_layout_passes=False, use_tc_tiling_on_sc=False),
    )
    out = score(ghe, ght, gte, gtt, rel_emb, rel_tr,
                pr.reshape(NW, NCH, CH))
    return out.reshape(B, 1)


def kernel(predict_h, predict_t, predict_r, ent_embeddings, rel_embeddings,
           ent_transfer, rel_transfer):
    return _run(predict_h, predict_t, predict_r, ent_embeddings,
                rel_embeddings, ent_transfer, rel_transfer)


# speedup vs baseline: 1.1553x; 1.1499x over previous
import jax
import jax.numpy as jnp
from jax import lax
from jax.experimental import pallas as pl
from jax.experimental.pallas import tpu as pltpu
from jax.experimental.pallas import tpu_sc as plsc

N_ENT = 1000000
N_REL = 1000
DIM = 64
B = 16384

NC = 2
NS = 16
L = 16
NW = NC * NS
BPW = B // NW
CH = 128
NCH = BPW // CH
K = DIM // L

NG = (N_ENT + 127) // 128
GPW = NG // NW
GREM = NG - GPW * NW
NBK = 16
MCAP = B + L
SROWS = 64
FLUSH_AT = SROWS - L


def _bc(x):
    return lax.broadcast(x, (L,))


def _hsum(v):
    return _bc(jnp.sum(v))


def _rsqrt_vec(x):
    xi = lax.bitcast_convert_type(x, jnp.int32)
    yi = jnp.int32(0x5F3759DF) - lax.shift_right_arithmetic(xi, jnp.int32(1))
    y = lax.bitcast_convert_type(yi, jnp.float32)
    xh = x * 0.5
    for _ in range(3):
        y = y * (1.5 - xh * y * y)
    return y


def _pop(m):
    return plsc.all_reduce_population_count(m)[0]


def _gather_body(emb_t, tr_t, emb_tl, tr_tl, ph, pt, ghe, ght, gte, gtt,
                 idxv, mb_h, mb_t, w_bufs, stg_h, stg_t,
                 sidx_h, sidx_t, sems, ssem):
    c = lax.axis_index("c")
    s = lax.axis_index("s")
    w = s * NC + c
    lo = w * GPW + jnp.minimum(w, GREM)
    ngr = GPW + (w < GREM).astype(jnp.int32)

    lane = lax.iota(jnp.int32, L)

    def scan_list(src_hbm, mbuf):
        pltpu.sync_copy(src_hbm, idxv.at[pl.ds(0, B)])

        def body(v, cnt):
            e = idxv[pl.ds(v * L, L)]
            eg = lax.shift_right_logical(e, 7)
            egl = eg - lo
            m = (egl >= 0) & (egl < ngr)
            packed = ((egl << 21) | ((e & 127) << 14) | (v * L + lane))
            plsc.store_compressed(mbuf.at[pl.ds(cnt, L)], packed, mask=m)
            return cnt + _pop(m)

        return lax.fori_loop(0, B // L, body, jnp.int32(0))

    cnt_h = scan_list(ph, mb_h)
    cnt_t = scan_list(pt, mb_t)

    def bucketize(mbuf, cnt, dst):
        nv = (cnt + L - 1) >> 4

        def count_body(u, counts):
            pk = mbuf[pl.ds(u * L, L)]
            valid = (u * L + lane) < cnt
            bk = lax.shift_right_logical(pk, 25)
            new = []
            for b in range(NBK):
                new.append(counts[b] + _pop((bk == b) & valid))
            return tuple(new)

        counts = lax.fori_loop(0, nv, count_body,
                               tuple(jnp.int32(0) for _ in range(NBK)))
        starts = []
        acc = jnp.int32(0)
        for b in range(NBK):
            starts.append(acc)
            acc = acc + counts[b]

        def place_body(u, offs):
            pk = mbuf[pl.ds(u * L, L)]
            valid = (u * L + lane) < cnt
            bk = lax.shift_right_logical(pk, 25)
            new = []
            for b in range(NBK):
                m = (bk == b) & valid
                plsc.store_compressed(dst.at[pl.ds(offs[b], L)], pk, mask=m)
                new.append(offs[b] + _pop(m))
            return tuple(new)

        lax.fori_loop(0, nv, place_body, tuple(starts))
        return starts, counts

    st_h, ct_h = bucketize(mb_h, cnt_h, idxv)
    bk_h = idxv
    st_t, ct_t = bucketize(mb_t, cnt_t, mb_h)
    bk_t = mb_h

    def issue_window(gr, par):
        embw, trw = w_bufs[par]
        sem = sems[par]
        g = lo + gr
        col = pl.multiple_of(g * 128, 128)

        @pl.when(g < NG - 1)
        def _():
            pltpu.async_copy(emb_t.at[:, pl.ds(col, 128)], embw, sem)
            pltpu.async_copy(tr_t.at[:, pl.ds(col, 128)], trw, sem)

        @pl.when(g == NG - 1)
        def _():
            pltpu.async_copy(emb_tl, embw, sem)
            pltpu.async_copy(tr_tl, trw, sem)

    def wait_window(gr, par):
        embw, trw = w_bufs[par]
        sem = sems[par]
        pltpu.make_async_copy(emb_t.at[:, pl.ds(0, 128)], embw, sem).wait()
        pltpu.make_async_copy(emb_t.at[:, pl.ds(0, 128)], trw, sem).wait()

    def flush(stg, sidx, out_a, out_b, fill):
        for v in range(SROWS // L):
            cur = sidx[pl.ds(v * L, L)]
            pad = (v * L + lane) >= fill
            sidx[pl.ds(v * L, L)] = jnp.where(pad, B + lane, cur)
        sa, sb = stg
        ca = pltpu.async_copy(sa, out_a.at[sidx], ssem)
        cb = pltpu.async_copy(sb, out_b.at[sidx], ssem)
        ca.wait()
        cb.wait()

    def select16(vals, b):
        r = vals[0]
        for i in range(1, NBK):
            r = jnp.where(b == i, vals[i], r)
        return r

    def process_window(gr, par, bk, starts, counts, stg, sidx, out_a, out_b,
                       fill):
        embw, trw = w_bufs[par]
        bkt = lax.shift_right_logical(gr, 4)
        start = select16(starts, bkt)
        cntb = select16(counts, bkt)
        nv = (cntb + L - 1) >> 4

        def body(u, fill):
            off = start + u * L
            pk = bk[pl.ds(off, L)]
            valid = (off + lane) < (start + cntb)
            mm = (lax.shift_right_logical(pk, 21) == gr) & valid
            nm = _pop(mm)

            def matched(fill):
                ein = lax.shift_right_logical(pk, 14) & 127
                slot = pk & 16383
                mmi = jnp.where(mm, 1, 0)
                ranks = plsc.cumsum(mmi) - 1
                plsc.store_compressed(sidx.at[pl.ds(fill, L)], slot, mask=mm)
                sa, sb = stg
                for j in range(L):
                    @pl.when(mmi[j] > 0)
                    def _():
                        ej = _bc(ein[j])
                        row = fill + ranks[j]
                        for k in range(K):
                            fidx = lane + (k * L)
                            sa[row, pl.ds(k * L, L)] = plsc.load_gather(
                                embw, [fidx, ej])
                            sb[row, pl.ds(k * L, L)] = plsc.load_gather(
                                trw, [fidx, ej])
                return fill + nm

            fill = lax.cond(nm > 0, matched, lambda f: f, fill)

            def do_flush(f):
                flush(stg, sidx, out_a, out_b, f)
                return jnp.int32(0)

            fill = lax.cond(fill >= FLUSH_AT, do_flush, lambda f: f, fill)
            return fill

        return lax.fori_loop(0, nv, body, fill)

    issue_window(0, 0)
    npair = (ngr + 1) >> 1

    def pair_body(p, carry):
        fh, ft = carry
        g0 = 2 * p
        g1 = 2 * p + 1

        @pl.when(g1 < ngr)
        def _():
            issue_window(g1, 1)

        wait_window(g0, 0)
        fh = process_window(g0, 0, bk_h, st_h, ct_h, stg_h, sidx_h,
                            ghe, ght, fh)
        ft = process_window(g0, 0, bk_t, st_t, ct_t, stg_t, sidx_t,
                            gte, gtt, ft)

        @pl.when(g1 + 1 < ngr)
        def _():
            issue_window(g1 + 1, 0)

        def odd(carry):
            fh, ft = carry
            wait_window(g1, 1)
            fh = process_window(g1, 1, bk_h, st_h, ct_h, stg_h, sidx_h,
                                ghe, ght, fh)
            ft = process_window(g1, 1, bk_t, st_t, ct_t, stg_t, sidx_t,
                                gte, gtt, ft)
            return fh, ft

        return lax.cond(g1 < ngr, odd, lambda cc: cc, (fh, ft))

    fh, ft = lax.fori_loop(0, npair, pair_body, (jnp.int32(0), jnp.int32(0)))
    flush(stg_h, sidx_h, ghe, ght, fh)
    flush(stg_t, sidx_t, gte, gtt, ft)


def _score_body(ghe, ght, gte, gtt, rel_emb, rel_tr, pr, out,
                idx_r, hb, htb, tb, ttb, rb, rtb, outv, sem):
    c = lax.axis_index("c")
    s = lax.axis_index("s")
    wid = s * NC + c
    base = wid * BPW

    pltpu.sync_copy(pr.at[wid], idx_r)

    lane = lax.iota(jnp.int32, L)
    mask0 = lane == 0

    for ci in range(NCH):
        row0 = base + ci * CH
        cps = [
            pltpu.async_copy(ghe.at[pl.ds(row0, CH), pl.ds(0, DIM)], hb, sem),
            pltpu.async_copy(ght.at[pl.ds(row0, CH), pl.ds(0, DIM)], htb, sem),
            pltpu.async_copy(gte.at[pl.ds(row0, CH), pl.ds(0, DIM)], tb, sem),
            pltpu.async_copy(gtt.at[pl.ds(row0, CH), pl.ds(0, DIM)], ttb, sem),
            pltpu.async_copy(rel_emb.at[idx_r.at[ci]], rb, sem),
            pltpu.async_copy(rel_tr.at[idx_r.at[ci]], rtb, sem),
        ]
        for cp in cps:
            cp.wait()

        @plsc.parallel_loop(0, CH, 1, unroll=2)
        def row_body(i):
            h = [hb[i, pl.ds(L * k, L)] for k in range(K)]
            ht = [htb[i, pl.ds(L * k, L)] for k in range(K)]
            t = [tb[i, pl.ds(L * k, L)] for k in range(K)]
            tt = [ttb[i, pl.ds(L * k, L)] for k in range(K)]
            r = [rb[i, pl.ds(L * k, L)] for k in range(K)]
            rt = [rtb[i, pl.ds(L * k, L)] for k in range(K)]

            def dot(a, b):
                p = a[0] * b[0]
                for k in range(1, K):
                    p = p + a[k] * b[k]
                return _hsum(p)

            d_hht = dot(h, ht)
            d_hrt = dot(h, rt)
            d_rtrt = dot(rt, rt)
            d_hh = dot(h, h)
            d_ttt = dot(t, tt)
            d_trt = dot(t, rt)
            d_tt = dot(t, t)

            n_h = d_hh + d_hht * (2.0 * d_hrt + d_hht * d_rtrt)
            n_t = d_tt + d_ttt * (2.0 * d_trt + d_ttt * d_rtrt)
            rih = _rsqrt_vec(jnp.maximum(n_h, 1e-12))
            rit = _rsqrt_vec(jnp.maximum(n_t, 1e-12))

            bmix = d_hht * rih - d_ttt * rit
            q = jnp.abs(h[0] * rih + rt[0] * bmix + r[0] - t[0] * rit)
            for k in range(1, K):
                q = q + jnp.abs(h[k] * rih + rt[k] * bmix + r[k] - t[k] * rit)
            plsc.store_scatter(
                outv, [_bc(jnp.int32(ci * CH) + i)], _hsum(q), mask=mask0)

    pltpu.sync_copy(outv, out.at[wid])


@jax.jit
def _run(ph, pt, pr, ent_emb, rel_emb, ent_tr, rel_tr):
    mesh = plsc.VectorSubcoreMesh(core_axis_name="c", subcore_axis_name="s")
    gshape = jax.ShapeDtypeStruct((B + L, 128), jnp.float32)
    gather = pl.kernel(
        _gather_body,
        out_type=(gshape, gshape, gshape, gshape),
        mesh=mesh,
        scratch_types=[
            pltpu.VMEM((MCAP,), jnp.int32),
            pltpu.VMEM((MCAP,), jnp.int32),
            pltpu.VMEM((MCAP,), jnp.int32),
            [[pltpu.VMEM((DIM, 128), jnp.float32) for _ in range(2)]
             for _ in range(2)],
            [pltpu.VMEM((SROWS, 128), jnp.float32) for _ in range(2)],
            [pltpu.VMEM((SROWS, 128), jnp.float32) for _ in range(2)],
            pltpu.VMEM((SROWS,), jnp.int32),
            pltpu.VMEM((SROWS,), jnp.int32),
            [pltpu.SemaphoreType.DMA, pltpu.SemaphoreType.DMA],
            pltpu.SemaphoreType.DMA,
        ],
        compiler_params=pltpu.CompilerParams(
            needs_layout_passes=False, use_tc_tiling_on_sc=True),
    )
    emb_view = ent_emb.T
    tr_view = ent_tr.T
    ghe, ght, gte, gtt = gather(
        emb_view, tr_view,
        lax.pad(lax.slice(emb_view, (0, N_ENT - 64), (DIM, N_ENT)),
                0.0, ((0, 0, 0), (0, 64, 0))),
        lax.pad(lax.slice(tr_view, (0, N_ENT - 64), (DIM, N_ENT)),
                0.0, ((0, 0, 0), (0, 64, 0))),
        ph, pt)

    score = pl.kernel(
        _score_body,
        out_type=jax.ShapeDtypeStruct((NW, BPW), jnp.float32),
        mesh=mesh,
        scratch_types=[
            pltpu.VMEM((NCH, CH), jnp.int32),
            pltpu.VMEM((CH, DIM), jnp.float32),
            pltpu.VMEM((CH, DIM), jnp.float32),
            pltpu.VMEM((CH, DIM), jnp.float32),
            pltpu.VMEM((CH, DIM), jnp.float32),
            pltpu.VMEM((CH, DIM), jnp.float32),
            pltpu.VMEM((CH, DIM), jnp.float32),
            pltpu.VMEM((BPW,), jnp.float32),
            pltpu.SemaphoreType.DMA,
        ],
        compiler_params=pltpu.CompilerParams(
            needs_layout_passes=False, use_tc_tiling_on_sc=False),
    )
    out = score(ghe, ght, gte, gtt, rel_emb, rel_tr,
                pr.reshape(NW, NCH, CH))
    return out.reshape(B, 1)


def kernel(predict_h, predict_t, predict_r, ent_embeddings, rel_embeddings,
           ent_transfer, rel_transfer):
    return _run(predict_h, predict_t, predict_r, ent_embeddings,
                rel_embeddings, ent_transfer, rel_transfer)
